# baseline (speedup 1.0000x reference)
import jax
import jax.numpy as jnp
from jax.experimental import pallas as pl
from jax.experimental.pallas import tpu as pltpu

N = 10000
N_USER = 6000
D = 64
BM = 400
NBLK = N // BM
BM2 = 1000
NBLK2 = N // BM2
UBLK2 = N_USER // BM2
ALPHA = 0.25


def _pass1_kernel(a_ref, u_in, v_in, r1_ref, a16_ref, x_ref):
    t = pl.program_id(0)
    i = jnp.maximum(t - 1, 0)
    rows = pl.ds(i * BM, BM)

    @pl.when(t == 0)
    def _assemble_x():
        x_ref[:N_USER, :] = u_in[...]
        x_ref[N_USER:, :] = v_in[...]

    @pl.when(t > 0)
    def _layer0():
        a = a_ref[...]
        a16_ref[...] = a.astype(jnp.bfloat16)
        r1_ref[...] = ALPHA * jnp.dot(
            a, x_ref[...], preferred_element_type=jnp.float32
        ) + ALPHA * x_ref[rows, :]


def _pass2_kernel(a16_ref, r1_in, u_in, v_in, u_ref, v_ref, rb_ref, r2_ref):
    t = pl.program_id(0)
    l = t // NBLK2
    i = t % NBLK2
    rows = pl.ds(i * BM2, BM2)
    urows = pl.ds(i * BM2, BM2)
    vrows = pl.ds(i * BM2 - N_USER, BM2)

    @pl.when(t == 0)
    def _cast_r1():
        rb_ref[...] = r1_in[...].astype(jnp.bfloat16)

    @pl.when(jnp.logical_and(l == 0, i < UBLK2))
    def _layer1_user():
        r2_ref[rows, :] = (jnp.dot(a16_ref[...], rb_ref[...],
                                   preferred_element_type=jnp.float32)
                           + ALPHA * u_in[urows, :]).astype(jnp.bfloat16)

    @pl.when(jnp.logical_and(l == 0, i >= UBLK2))
    def _layer1_item():
        r2_ref[rows, :] = (jnp.dot(a16_ref[...], rb_ref[...],
                                   preferred_element_type=jnp.float32)
                           + ALPHA * v_in[vrows, :]).astype(jnp.bfloat16)

    @pl.when(jnp.logical_and(l == 1, i < UBLK2))
    def _layer2_user():
        u_ref[...] = jnp.dot(a16_ref[...], r2_ref[...],
                             preferred_element_type=jnp.float32
                             ) + ALPHA * u_in[urows, :]

    @pl.when(jnp.logical_and(l == 1, i >= UBLK2))
    def _layer2_item():
        v_ref[...] = jnp.dot(a16_ref[...], r2_ref[...],
                             preferred_element_type=jnp.float32
                             ) + ALPHA * v_in[vrows, :]


def kernel(user_embedding, item_embedding, norm_adj):
    r1, a16 = pl.pallas_call(
        _pass1_kernel,
        grid=(1 + NBLK,),
        in_specs=[
            pl.BlockSpec((BM, N), lambda t: (jnp.maximum(t - 1, 0), 0)),
            pl.BlockSpec((N_USER, D), lambda t: (0, 0)),
            pl.BlockSpec((N - N_USER, D), lambda t: (0, 0)),
        ],
        out_specs=[
            pl.BlockSpec((BM, D), lambda t: (jnp.maximum(t - 1, 0), 0)),
            pl.BlockSpec((BM, N), lambda t: (jnp.maximum(t - 1, 0), 0)),
        ],
        out_shape=[
            jax.ShapeDtypeStruct((N, D), jnp.float32),
            jax.ShapeDtypeStruct((N, N), jnp.bfloat16),
        ],
        scratch_shapes=[pltpu.VMEM((N, D), jnp.float32)],
        compiler_params=pltpu.CompilerParams(
            dimension_semantics=("arbitrary",)),
    )(norm_adj, user_embedding, item_embedding)

    def _u_map(t):
        l, i = t // NBLK2, t % NBLK2
        return (jnp.where(l == 1, jnp.minimum(i, UBLK2 - 1), 0), 0)

    def _v_map(t):
        l, i = t // NBLK2, t % NBLK2
        return (jnp.where(l == 1, jnp.maximum(i - UBLK2, 0), 0), 0)

    u_out, v_out = pl.pallas_call(
        _pass2_kernel,
        grid=(2 * NBLK2,),
        in_specs=[
            pl.BlockSpec((BM2, N), lambda t: (t % NBLK2, 0)),
            pl.BlockSpec((N, D), lambda t: (0, 0)),
            pl.BlockSpec((N_USER, D), lambda t: (0, 0)),
            pl.BlockSpec((N - N_USER, D), lambda t: (0, 0)),
        ],
        out_specs=[
            pl.BlockSpec((BM2, D), _u_map),
            pl.BlockSpec((BM2, D), _v_map),
        ],
        out_shape=[
            jax.ShapeDtypeStruct((N_USER, D), jnp.float32),
            jax.ShapeDtypeStruct((N - N_USER, D), jnp.float32),
        ],
        scratch_shapes=[
            pltpu.VMEM((N, D), jnp.bfloat16),
            pltpu.VMEM((N, D), jnp.bfloat16),
        ],
        compiler_params=pltpu.CompilerParams(
            dimension_semantics=("arbitrary",)),
    )(a16, r1, user_embedding, item_embedding)
    return (u_out, v_out)


# speedup vs baseline: 1.0043x; 1.0043x over previous
import jax
import jax.numpy as jnp
from jax.experimental import pallas as pl
from jax.experimental.pallas import tpu as pltpu

N = 10000
N_USER = 6000
D = 64
BM = 400
NBLK = N // BM
BM2 = 1000
NBLK2 = N // BM2
UBLK2 = N_USER // BM2
ALPHA = 0.25


def _pass1_kernel(a_ref, u_in, v_in, r1_ref, a16_ref, x_ref):
    t = pl.program_id(0)
    i = jnp.maximum(t - 1, 0)
    rows = pl.ds(i * BM, BM)

    @pl.when(t == 0)
    def _assemble_x():
        x_ref[:N_USER, :] = u_in[...]
        x_ref[N_USER:, :] = v_in[...]

    @pl.when(t > 0)
    def _layer0():
        a = a_ref[...]
        a16_ref[...] = a.astype(jnp.bfloat16)
        r1_ref[...] = (ALPHA * jnp.dot(
            a, x_ref[...], preferred_element_type=jnp.float32
        ) + ALPHA * x_ref[rows, :]).astype(jnp.bfloat16)


def _pass2_kernel(a16_ref, r1_in, u_in, v_in, u_ref, v_ref, r2_ref):
    t = pl.program_id(0)
    l = t // NBLK2
    i = t % NBLK2
    rows = pl.ds(i * BM2, BM2)
    urows = pl.ds(i * BM2, BM2)
    vrows = pl.ds(i * BM2 - N_USER, BM2)

    @pl.when(jnp.logical_and(l == 0, i < UBLK2))
    def _layer1_user():
        r2_ref[rows, :] = (jnp.dot(a16_ref[...], r1_in[...],
                                   preferred_element_type=jnp.float32)
                           + ALPHA * u_in[urows, :]).astype(jnp.bfloat16)

    @pl.when(jnp.logical_and(l == 0, i >= UBLK2))
    def _layer1_item():
        r2_ref[rows, :] = (jnp.dot(a16_ref[...], r1_in[...],
                                   preferred_element_type=jnp.float32)
                           + ALPHA * v_in[vrows, :]).astype(jnp.bfloat16)

    @pl.when(jnp.logical_and(l == 1, i < UBLK2))
    def _layer2_user():
        u_ref[...] = jnp.dot(a16_ref[...], r2_ref[...],
                             preferred_element_type=jnp.float32
                             ) + ALPHA * u_in[urows, :]

    @pl.when(jnp.logical_and(l == 1, i >= UBLK2))
    def _layer2_item():
        v_ref[...] = jnp.dot(a16_ref[...], r2_ref[...],
                             preferred_element_type=jnp.float32
                             ) + ALPHA * v_in[vrows, :]


def kernel(user_embedding, item_embedding, norm_adj):
    r1, a16 = pl.pallas_call(
        _pass1_kernel,
        grid=(1 + NBLK,),
        in_specs=[
            pl.BlockSpec((BM, N), lambda t: (jnp.maximum(t - 1, 0), 0)),
            pl.BlockSpec((N_USER, D), lambda t: (0, 0)),
            pl.BlockSpec((N - N_USER, D), lambda t: (0, 0)),
        ],
        out_specs=[
            pl.BlockSpec((BM, D), lambda t: (jnp.maximum(t - 1, 0), 0)),
            pl.BlockSpec((BM, N), lambda t: (jnp.maximum(t - 1, 0), 0)),
        ],
        out_shape=[
            jax.ShapeDtypeStruct((N, D), jnp.bfloat16),
            jax.ShapeDtypeStruct((N, N), jnp.bfloat16),
        ],
        scratch_shapes=[pltpu.VMEM((N, D), jnp.float32)],
        compiler_params=pltpu.CompilerParams(
            dimension_semantics=("arbitrary",)),
    )(norm_adj, user_embedding, item_embedding)

    def _u_map(t):
        l, i = t // NBLK2, t % NBLK2
        return (jnp.where(l == 1, jnp.minimum(i, UBLK2 - 1), 0), 0)

    def _v_map(t):
        l, i = t // NBLK2, t % NBLK2
        return (jnp.where(l == 1, jnp.maximum(i - UBLK2, 0), 0), 0)

    u_out, v_out = pl.pallas_call(
        _pass2_kernel,
        grid=(2 * NBLK2,),
        in_specs=[
            pl.BlockSpec((BM2, N), lambda t: (t % NBLK2, 0)),
            pl.BlockSpec((N, D), lambda t: (0, 0)),
            pl.BlockSpec((N_USER, D), lambda t: (0, 0)),
            pl.BlockSpec((N - N_USER, D), lambda t: (0, 0)),
        ],
        out_specs=[
            pl.BlockSpec((BM2, D), _u_map),
            pl.BlockSpec((BM2, D), _v_map),
        ],
        out_shape=[
            jax.ShapeDtypeStruct((N_USER, D), jnp.float32),
            jax.ShapeDtypeStruct((N - N_USER, D), jnp.float32),
        ],
        scratch_shapes=[
            pltpu.VMEM((N, D), jnp.bfloat16),
        ],
        compiler_params=pltpu.CompilerParams(
            dimension_semantics=("arbitrary",)),
    )(a16, r1, user_embedding, item_embedding)
    return (u_out, v_out)
